# baseline (speedup 1.0000x reference)
import jax
import jax.numpy as jnp
from jax import lax
from jax.experimental import pallas as pl
from jax.experimental.pallas import tpu as pltpu
from jax.experimental.pallas import tpu_sc as plsc

N = 10000
E = 320000
D = 128

NC = 2
NS = 16
TILE_E = E // NS
CHUNK = 80
NCH = TILE_E // CHUNK
BCH = 25
NBLK = NCH // BCH
NPAD = 10240
ROWS_PT = NPAD // NS


def _sc_body(nodes_hbm, eidx_hbm, zrow_hbm, zcnt_hbm,
             acc1_out, acc2_out, cnt_out,
             acc_s, cnt_s, gl, sl, buf0, buf1, buf2, ones, sem0, sem1,
             sem2):
    c = lax.axis_index("c")
    s = lax.axis_index("s")

    for i in range(CHUNK // 16):
        ones[pl.ds(i * 16, 16)] = jnp.ones((16,), jnp.float32)

    pltpu.sync_copy(zrow_hbm, acc_s.at[pl.ds(s * ROWS_PT, ROWS_PT)])
    pltpu.sync_copy(zcnt_hbm, cnt_s.at[pl.ds(s * ROWS_PT, ROWS_PT)])
    plsc.subcore_barrier()

    bufs = (buf0, buf1, buf2)
    sems = (sem0, sem1, sem2)

    def gath(j, p):
        pltpu.async_copy(nodes_hbm.at[gl.at[j]], bufs[p], sems[p])

    def wait_g(p):
        pltpu.make_async_copy(nodes_hbm.at[gl.at[0]], bufs[p], sems[p]).wait()

    def scat(j, p):
        pltpu.sync_copy(bufs[p], acc_s.at[sl.at[j]], add=True)
        pltpu.sync_copy(ones, cnt_s.at[sl.at[j]], add=True)

    def block(b, carry):
        pltpu.sync_copy(eidx_hbm.at[c, s, pl.ds(b * BCH, BCH)], gl)
        pltpu.sync_copy(eidx_hbm.at[1 - c, s, pl.ds(b * BCH, BCH)], sl)
        gath(0, 0)
        gath(1, 1)

        def inner(t, carry2):
            j = 3 * t
            wait_g(0)
            gath(j + 2, 2)
            scat(j, 0)
            wait_g(1)
            gath(j + 3, 0)
            scat(j + 1, 1)
            wait_g(2)
            gath(jnp.minimum(j + 4, BCH - 1), 1)
            scat(j + 2, 2)
            return carry2

        lax.fori_loop(0, (BCH - 1) // 3, inner, 0)
        wait_g(0)
        scat(BCH - 1, 0)
        wait_g(1)
        return carry

    lax.fori_loop(0, NBLK, block, 0)
    plsc.subcore_barrier()

    sl_pt = pl.ds(s * ROWS_PT, ROWS_PT)

    @pl.when(c == 0)
    def _():
        pltpu.sync_copy(acc_s.at[sl_pt], acc1_out.at[sl_pt])

    @pl.when(c == 1)
    def _():
        pltpu.sync_copy(acc_s.at[sl_pt], acc2_out.at[sl_pt])

    pltpu.sync_copy(cnt_s.at[sl_pt], cnt_out.at[c, sl_pt])


_sc_aggregate = pl.kernel(
    _sc_body,
    out_type=(
        jax.ShapeDtypeStruct((NPAD, D), jnp.float32),
        jax.ShapeDtypeStruct((NPAD, D), jnp.float32),
        jax.ShapeDtypeStruct((NC, NPAD), jnp.float32),
    ),
    mesh=plsc.VectorSubcoreMesh(core_axis_name="c", subcore_axis_name="s"),
    scratch_types=[
        pltpu.VMEM_SHARED((NPAD, D), jnp.float32),
        pltpu.VMEM_SHARED((NPAD,), jnp.float32),
        pltpu.VMEM((BCH, CHUNK), jnp.int32),
        pltpu.VMEM((BCH, CHUNK), jnp.int32),
        pltpu.VMEM((CHUNK, D), jnp.float32),
        pltpu.VMEM((CHUNK, D), jnp.float32),
        pltpu.VMEM((CHUNK, D), jnp.float32),
        pltpu.VMEM((CHUNK,), jnp.float32),
        pltpu.SemaphoreType.DMA,
        pltpu.SemaphoreType.DMA,
        pltpu.SemaphoreType.DMA,
    ],
    compiler_params=pltpu.CompilerParams(use_tc_tiling_on_sc=False),
)


ROWS_TC = 400


def _tc_self_body(nodes_ref, w0_ref, out_ref):
    dn = (((1,), (1,)), ((), ()))
    out_ref[...] = lax.dot_general(nodes_ref[...], w0_ref[...], dn,
                                   preferred_element_type=jnp.float32)


def _tc_self(nodes, w0):
    row_spec = pl.BlockSpec((ROWS_TC, D), lambda i: (i, 0))
    w_spec = pl.BlockSpec((D, D), lambda i: (0, 0))
    return pl.pallas_call(
        _tc_self_body,
        grid=(N // ROWS_TC,),
        in_specs=[row_spec, w_spec],
        out_specs=row_spec,
        out_shape=jax.ShapeDtypeStruct((N, D), jnp.float32),
    )(nodes, w0)


def _tc_body(self_ref, a1_ref, a2_ref, cnt_ref, w1_ref, w2_ref, out_ref):
    dn = (((1,), (1,)), ((), ()))
    cnt = jnp.sum(cnt_ref[...], axis=1, keepdims=True)
    denom = jnp.maximum(cnt, 1.0)
    msg = (lax.dot_general(a1_ref[...], w1_ref[...], dn,
                           preferred_element_type=jnp.float32)
           + lax.dot_general(a2_ref[...], w2_ref[...], dn,
                             preferred_element_type=jnp.float32)) / denom
    out_ref[...] = jnp.maximum(self_ref[...] + msg, 0.0)


def _tc_combine(self_t, a1, a2, cnt2, w1, w2):
    row_spec = pl.BlockSpec((ROWS_TC, D), lambda i: (i, 0))
    w_spec = pl.BlockSpec((D, D), lambda i: (0, 0))
    return pl.pallas_call(
        _tc_body,
        grid=(N // ROWS_TC,),
        in_specs=[row_spec, row_spec, row_spec,
                  pl.BlockSpec((ROWS_TC, NC), lambda i: (i, 0)),
                  w_spec, w_spec],
        out_specs=row_spec,
        out_shape=jax.ShapeDtypeStruct((N, D), jnp.float32),
    )(self_t, a1, a2, cnt2, w1, w2)


def kernel(nodes, edges, W0, W1, W2):
    eidx = edges.astype(jnp.int32).reshape(NC, NS, NBLK * BCH, CHUNK)
    zrow = jnp.zeros((ROWS_PT, D), jnp.float32)
    zcnt = jnp.zeros((ROWS_PT,), jnp.float32)
    self_t = _tc_self(nodes, W0)
    a1, a2, cnt = _sc_aggregate(nodes, eidx, zrow, zcnt)
    return _tc_combine(self_t, a1, a2, cnt.T, W1, W2)


# speedup vs baseline: 1.0094x; 1.0094x over previous
import jax
import jax.numpy as jnp
from jax import lax
from jax.experimental import pallas as pl
from jax.experimental.pallas import tpu as pltpu
from jax.experimental.pallas import tpu_sc as plsc

N = 10000
E = 320000
D = 128

NC = 2
NS = 16
TILE_E = E // NS
CHUNK = 80
NCH = TILE_E // CHUNK
BCH = 25
NBLK = NCH // BCH
NPAD = 10240
ROWS_PT = NPAD // NS


def _sc_body(nodes_hbm, eidx_hbm, zrow_hbm, zcnt_hbm,
             acc1_out, acc2_out, cnt_out,
             acc_s, cnt_s, gl, sl, buf0, buf1, buf2, ones, sem0, sem1,
             sem2):
    c = lax.axis_index("c")
    s = lax.axis_index("s")

    for i in range(CHUNK // 16):
        ones[pl.ds(i * 16, 16)] = jnp.ones((16,), jnp.float32)

    pltpu.sync_copy(zrow_hbm, acc_s.at[pl.ds(s * ROWS_PT, ROWS_PT)])
    pltpu.sync_copy(zcnt_hbm, cnt_s.at[pl.ds(s * ROWS_PT, ROWS_PT)])
    plsc.subcore_barrier()

    bufs = (buf0, buf1, buf2)
    sems = (sem0, sem1, sem2)

    def gath(j, p):
        pltpu.async_copy(nodes_hbm.at[gl.at[j]], bufs[p], sems[p])

    def wait_g(p):
        pltpu.make_async_copy(nodes_hbm.at[gl.at[0]], bufs[p], sems[p]).wait()

    def scat(j, p):
        pltpu.sync_copy(bufs[p], acc_s.at[sl.at[j]], add=True)
        pltpu.sync_copy(ones, cnt_s.at[sl.at[j]], add=True)

    def block(b, carry):
        pltpu.sync_copy(eidx_hbm.at[c, s, pl.ds(b * BCH, BCH)], gl)
        pltpu.sync_copy(eidx_hbm.at[1 - c, s, pl.ds(b * BCH, BCH)], sl)
        gath(0, 0)
        gath(1, 1)

        def inner(t, carry2):
            j = 3 * t
            wait_g(0)
            gath(j + 2, 2)
            scat(j, 0)
            wait_g(1)
            gath(j + 3, 0)
            scat(j + 1, 1)
            wait_g(2)
            gath(jnp.minimum(j + 4, BCH - 1), 1)
            scat(j + 2, 2)
            return carry2

        lax.fori_loop(0, (BCH - 1) // 3, inner, 0)
        wait_g(0)
        scat(BCH - 1, 0)
        wait_g(1)
        return carry

    lax.fori_loop(0, NBLK, block, 0)
    plsc.subcore_barrier()

    sl_pt = pl.ds(s * ROWS_PT, ROWS_PT)

    @pl.when(c == 0)
    def _():
        pltpu.sync_copy(acc_s.at[sl_pt], acc1_out.at[sl_pt])

    @pl.when(c == 1)
    def _():
        pltpu.sync_copy(acc_s.at[sl_pt], acc2_out.at[sl_pt])

    pltpu.sync_copy(cnt_s.at[sl_pt], cnt_out.at[c, sl_pt])


_sc_aggregate = pl.kernel(
    _sc_body,
    out_type=(
        jax.ShapeDtypeStruct((NPAD, D), jnp.float32),
        jax.ShapeDtypeStruct((NPAD, D), jnp.float32),
        jax.ShapeDtypeStruct((NC, NPAD), jnp.float32),
    ),
    mesh=plsc.VectorSubcoreMesh(core_axis_name="c", subcore_axis_name="s"),
    scratch_types=[
        pltpu.VMEM_SHARED((NPAD, D), jnp.float32),
        pltpu.VMEM_SHARED((NPAD,), jnp.float32),
        pltpu.VMEM((BCH, CHUNK), jnp.int32),
        pltpu.VMEM((BCH, CHUNK), jnp.int32),
        pltpu.VMEM((CHUNK, D), jnp.float32),
        pltpu.VMEM((CHUNK, D), jnp.float32),
        pltpu.VMEM((CHUNK, D), jnp.float32),
        pltpu.VMEM((CHUNK,), jnp.float32),
        pltpu.SemaphoreType.DMA,
        pltpu.SemaphoreType.DMA,
        pltpu.SemaphoreType.DMA,
    ],
    compiler_params=pltpu.CompilerParams(use_tc_tiling_on_sc=False),
)


ROWS_TC = 400


def _tc_body(nodes_ref, a1_ref, a2_ref, cnt_ref, w0_ref, w1_ref, w2_ref,
             out_ref):
    dn = (((1,), (1,)), ((), ()))
    a1 = a1_ref[...].reshape(ROWS_TC, D)
    a2 = a2_ref[...].reshape(ROWS_TC, D)
    cnt = jnp.sum(cnt_ref[...], axis=1, keepdims=True)
    denom = jnp.maximum(cnt, 1.0)
    msg = (lax.dot_general(a1, w1_ref[...], dn,
                           preferred_element_type=jnp.float32)
           + lax.dot_general(a2, w2_ref[...], dn,
                             preferred_element_type=jnp.float32)) / denom
    self_t = lax.dot_general(nodes_ref[...], w0_ref[...], dn,
                             preferred_element_type=jnp.float32)
    out_ref[...] = jnp.maximum(self_t + msg, 0.0)


def _tc_combine(nodes, a1, a2, cnt2, w0, w1, w2):
    row_spec = pl.BlockSpec((ROWS_TC, D), lambda i: (i, 0))
    w_spec = pl.BlockSpec((D, D), lambda i: (0, 0))
    return pl.pallas_call(
        _tc_body,
        grid=(N // ROWS_TC,),
        in_specs=[row_spec,
                  pl.BlockSpec((ROWS_TC * D,), lambda i: (i,)),
                  pl.BlockSpec((ROWS_TC * D,), lambda i: (i,)),
                  pl.BlockSpec((ROWS_TC, NC), lambda i: (i, 0)),
                  w_spec, w_spec, w_spec],
        out_specs=row_spec,
        out_shape=jax.ShapeDtypeStruct((N, D), jnp.float32),
    )(nodes, a1, a2, cnt2, w0, w1, w2)


def kernel(nodes, edges, W0, W1, W2):
    eidx = edges.astype(jnp.int32).reshape(NC, NS, NBLK * BCH, CHUNK)
    zrow = jnp.zeros((ROWS_PT, D), jnp.float32)
    zcnt = jnp.zeros((ROWS_PT,), jnp.float32)
    a1, a2, cnt = _sc_aggregate(nodes, eidx, zrow, zcnt)
    a1 = a1.reshape(NPAD * D)
    a2 = a2.reshape(NPAD * D)
    return _tc_combine(nodes, a1, a2, cnt.T, W0, W1, W2)


# speedup vs baseline: 1.0287x; 1.0191x over previous
import jax
import jax.numpy as jnp
from jax import lax
from jax.experimental import pallas as pl
from jax.experimental.pallas import tpu as pltpu
from jax.experimental.pallas import tpu_sc as plsc

N = 10000
E = 320000
D = 128

NC = 2
NS = 16
TILE_E = E // NS
CHUNK = 80
NCH = TILE_E // CHUNK
BCH = 25
NBLK = NCH // BCH
NPAD = 10240
ROWS_PT = NPAD // NS


def _sc_body(nodes_hbm, eidx_hbm,
             acc1_out, acc2_out, cnt_out,
             acc_s, cnt_s, gl, sl, buf0, buf1, buf2, ones, zcnt, sem0,
             sem1, sem2):
    c = lax.axis_index("c")
    s = lax.axis_index("s")

    for i in range(CHUNK // 16):
        ones[pl.ds(i * 16, 16)] = jnp.ones((16,), jnp.float32)

    zero16 = jnp.zeros((16,), jnp.float32)
    for r in range(CHUNK):
        for i in range(D // 16):
            buf0[r, pl.ds(i * 16, 16)] = zero16
    for i in range(ROWS_PT // 16):
        zcnt[pl.ds(i * 16, 16)] = zero16
    for r in range(ROWS_PT // CHUNK):
        pltpu.sync_copy(
            buf0, acc_s.at[pl.ds(s * ROWS_PT + r * CHUNK, CHUNK)])
    pltpu.sync_copy(zcnt, cnt_s.at[pl.ds(s * ROWS_PT, ROWS_PT)])
    plsc.subcore_barrier()

    bufs = (buf0, buf1, buf2)
    sems = (sem0, sem1, sem2)

    def gath(j, p):
        pltpu.async_copy(nodes_hbm.at[gl.at[j]], bufs[p], sems[p])

    def wait_g(p):
        pltpu.make_async_copy(nodes_hbm.at[gl.at[0]], bufs[p], sems[p]).wait()

    def scat(j, p):
        pltpu.sync_copy(bufs[p], acc_s.at[sl.at[j]], add=True)
        pltpu.sync_copy(ones, cnt_s.at[sl.at[j]], add=True)

    def block(b, carry):
        pltpu.sync_copy(eidx_hbm.at[c, s, pl.ds(b * BCH, BCH)], gl)
        pltpu.sync_copy(eidx_hbm.at[1 - c, s, pl.ds(b * BCH, BCH)], sl)
        gath(0, 0)
        gath(1, 1)

        def inner(t, carry2):
            j = 3 * t
            wait_g(0)
            gath(j + 2, 2)
            scat(j, 0)
            wait_g(1)
            gath(j + 3, 0)
            scat(j + 1, 1)
            wait_g(2)
            gath(jnp.minimum(j + 4, BCH - 1), 1)
            scat(j + 2, 2)
            return carry2

        lax.fori_loop(0, (BCH - 1) // 3, inner, 0)
        wait_g(0)
        scat(BCH - 1, 0)
        wait_g(1)
        return carry

    lax.fori_loop(0, NBLK, block, 0)
    plsc.subcore_barrier()

    sl_pt = pl.ds(s * ROWS_PT, ROWS_PT)

    @pl.when(c == 0)
    def _():
        pltpu.sync_copy(acc_s.at[sl_pt], acc1_out.at[sl_pt])

    @pl.when(c == 1)
    def _():
        pltpu.sync_copy(acc_s.at[sl_pt], acc2_out.at[sl_pt])

    pltpu.sync_copy(cnt_s.at[sl_pt], cnt_out.at[c, sl_pt])


_sc_aggregate = pl.kernel(
    _sc_body,
    out_type=(
        jax.ShapeDtypeStruct((NPAD, D), jnp.float32),
        jax.ShapeDtypeStruct((NPAD, D), jnp.float32),
        jax.ShapeDtypeStruct((NC, NPAD), jnp.float32),
    ),
    mesh=plsc.VectorSubcoreMesh(core_axis_name="c", subcore_axis_name="s"),
    scratch_types=[
        pltpu.VMEM_SHARED((NPAD, D), jnp.float32),
        pltpu.VMEM_SHARED((NPAD,), jnp.float32),
        pltpu.VMEM((BCH, CHUNK), jnp.int32),
        pltpu.VMEM((BCH, CHUNK), jnp.int32),
        pltpu.VMEM((CHUNK, D), jnp.float32),
        pltpu.VMEM((CHUNK, D), jnp.float32),
        pltpu.VMEM((CHUNK, D), jnp.float32),
        pltpu.VMEM((CHUNK,), jnp.float32),
        pltpu.VMEM((ROWS_PT,), jnp.float32),
        pltpu.SemaphoreType.DMA,
        pltpu.SemaphoreType.DMA,
        pltpu.SemaphoreType.DMA,
    ],
    compiler_params=pltpu.CompilerParams(use_tc_tiling_on_sc=False),
)


ROWS_TC = 400


def _tc_body(nodes_ref, a1_ref, a2_ref, cnt_ref, w0_ref, w1_ref, w2_ref,
             out_ref):
    dn = (((1,), (1,)), ((), ()))
    a1 = a1_ref[...].reshape(ROWS_TC, D)
    a2 = a2_ref[...].reshape(ROWS_TC, D)
    cnt = jnp.sum(cnt_ref[...], axis=1, keepdims=True)
    denom = jnp.maximum(cnt, 1.0)
    msg = (lax.dot_general(a1, w1_ref[...], dn,
                           preferred_element_type=jnp.float32)
           + lax.dot_general(a2, w2_ref[...], dn,
                             preferred_element_type=jnp.float32)) / denom
    self_t = lax.dot_general(nodes_ref[...], w0_ref[...], dn,
                             preferred_element_type=jnp.float32)
    out_ref[...] = jnp.maximum(self_t + msg, 0.0)


def _tc_combine(nodes, a1, a2, cnt2, w0, w1, w2):
    row_spec = pl.BlockSpec((ROWS_TC, D), lambda i: (i, 0))
    w_spec = pl.BlockSpec((D, D), lambda i: (0, 0))
    return pl.pallas_call(
        _tc_body,
        grid=(N // ROWS_TC,),
        in_specs=[row_spec,
                  pl.BlockSpec((ROWS_TC * D,), lambda i: (i,)),
                  pl.BlockSpec((ROWS_TC * D,), lambda i: (i,)),
                  pl.BlockSpec((ROWS_TC, NC), lambda i: (i, 0)),
                  w_spec, w_spec, w_spec],
        out_specs=row_spec,
        out_shape=jax.ShapeDtypeStruct((N, D), jnp.float32),
    )(nodes, a1, a2, cnt2, w0, w1, w2)


def kernel(nodes, edges, W0, W1, W2):
    eidx = edges.astype(jnp.int32).reshape(NC, NS, NBLK * BCH, CHUNK)
    a1, a2, cnt = _sc_aggregate(nodes, eidx)
    a1 = a1.reshape(NPAD * D)
    a2 = a2.reshape(NPAD * D)
    return _tc_combine(nodes, a1, a2, cnt.T, W0, W1, W2)


# speedup vs baseline: 1.0635x; 1.0338x over previous
import jax
import jax.numpy as jnp
from jax import lax
from jax.experimental import pallas as pl
from jax.experimental.pallas import tpu as pltpu
from jax.experimental.pallas import tpu_sc as plsc

N = 10000
E = 320000
D = 128

NC = 2
NS = 16
TILE_E = E // NS
CHUNK = 80
NCH = TILE_E // CHUNK
BCH = 25
NBLK = NCH // BCH
NPAD = 10240
ROWS_PT = NPAD // NS


def _sc_body(nodes_hbm, eidx_hbm,
             acc1_out, acc2_out, cnt_out,
             acc_s, cnt_s, gla, sla, glb, slb, buf0, buf1, buf2, ones,
             zcnt, sem0, sem1, sem2, semi, semj):
    c = lax.axis_index("c")
    s = lax.axis_index("s")

    for i in range(CHUNK // 16):
        ones[pl.ds(i * 16, 16)] = jnp.ones((16,), jnp.float32)

    zero16 = jnp.zeros((16,), jnp.float32)
    for r in range(CHUNK):
        for i in range(D // 16):
            buf0[r, pl.ds(i * 16, 16)] = zero16
    for i in range(ROWS_PT // 16):
        zcnt[pl.ds(i * 16, 16)] = zero16
    for r in range(ROWS_PT // CHUNK):
        pltpu.sync_copy(
            buf0, acc_s.at[pl.ds(s * ROWS_PT + r * CHUNK, CHUNK)])
    pltpu.sync_copy(zcnt, cnt_s.at[pl.ds(s * ROWS_PT, ROWS_PT)])
    plsc.subcore_barrier()

    bufs = (buf0, buf1, buf2)
    sems = (sem0, sem1, sem2)

    def proc(gl, sl):
        def gath(j, p):
            pltpu.async_copy(nodes_hbm.at[gl.at[j]], bufs[p], sems[p])

        def wait_g(p):
            pltpu.make_async_copy(
                nodes_hbm.at[gl.at[0]], bufs[p], sems[p]).wait()

        def scat(j, p):
            pltpu.sync_copy(bufs[p], acc_s.at[sl.at[j]], add=True)
            pltpu.sync_copy(ones, cnt_s.at[sl.at[j]], add=True)

        gath(0, 0)
        gath(1, 1)

        def inner(t, carry2):
            j = 3 * t
            wait_g(0)
            gath(j + 2, 2)
            scat(j, 0)
            wait_g(1)
            gath(j + 3, 0)
            scat(j + 1, 1)
            wait_g(2)
            gath(jnp.minimum(j + 4, BCH - 1), 1)
            scat(j + 2, 2)
            return carry2

        lax.fori_loop(0, (BCH - 1) // 3, inner, 0)
        wait_g(0)
        scat(BCH - 1, 0)
        wait_g(1)

    def load_idx(b, gl, sl, sem):
        pltpu.async_copy(eidx_hbm.at[c, s, pl.ds(b * BCH, BCH)], gl, sem)
        pltpu.async_copy(eidx_hbm.at[1 - c, s, pl.ds(b * BCH, BCH)], sl,
                         sem)

    def wait_idx(b, gl, sl, sem):
        pltpu.make_async_copy(
            eidx_hbm.at[c, s, pl.ds(0, BCH)], gl, sem).wait()
        pltpu.make_async_copy(
            eidx_hbm.at[c, s, pl.ds(0, BCH)], sl, sem).wait()

    load_idx(0, gla, sla, semi)

    def super_block(t, carry):
        ba = 2 * t
        wait_idx(ba, gla, sla, semi)
        load_idx(ba + 1, glb, slb, semj)
        proc(gla, sla)
        wait_idx(ba + 1, glb, slb, semj)
        load_idx(jnp.minimum(ba + 2, NBLK - 1), gla, sla, semi)
        proc(glb, slb)
        return carry

    lax.fori_loop(0, NBLK // 2, super_block, 0)
    pltpu.make_async_copy(eidx_hbm.at[c, s, pl.ds(0, BCH)], gla, semi).wait()
    pltpu.make_async_copy(eidx_hbm.at[c, s, pl.ds(0, BCH)], sla, semi).wait()
    plsc.subcore_barrier()

    sl_pt = pl.ds(s * ROWS_PT, ROWS_PT)

    @pl.when(c == 0)
    def _():
        pltpu.sync_copy(acc_s.at[sl_pt], acc1_out.at[sl_pt])

    @pl.when(c == 1)
    def _():
        pltpu.sync_copy(acc_s.at[sl_pt], acc2_out.at[sl_pt])

    pltpu.sync_copy(cnt_s.at[sl_pt], cnt_out.at[c, sl_pt])


_sc_aggregate = pl.kernel(
    _sc_body,
    out_type=(
        jax.ShapeDtypeStruct((NPAD, D), jnp.float32),
        jax.ShapeDtypeStruct((NPAD, D), jnp.float32),
        jax.ShapeDtypeStruct((NC, NPAD), jnp.float32),
    ),
    mesh=plsc.VectorSubcoreMesh(core_axis_name="c", subcore_axis_name="s"),
    scratch_types=[
        pltpu.VMEM_SHARED((NPAD, D), jnp.float32),
        pltpu.VMEM_SHARED((NPAD,), jnp.float32),
        pltpu.VMEM((BCH, CHUNK), jnp.int32),
        pltpu.VMEM((BCH, CHUNK), jnp.int32),
        pltpu.VMEM((BCH, CHUNK), jnp.int32),
        pltpu.VMEM((BCH, CHUNK), jnp.int32),
        pltpu.VMEM((CHUNK, D), jnp.float32),
        pltpu.VMEM((CHUNK, D), jnp.float32),
        pltpu.VMEM((CHUNK, D), jnp.float32),
        pltpu.VMEM((CHUNK,), jnp.float32),
        pltpu.VMEM((ROWS_PT,), jnp.float32),
        pltpu.SemaphoreType.DMA,
        pltpu.SemaphoreType.DMA,
        pltpu.SemaphoreType.DMA,
        pltpu.SemaphoreType.DMA,
        pltpu.SemaphoreType.DMA,
    ],
    compiler_params=pltpu.CompilerParams(use_tc_tiling_on_sc=False),
)


ROWS_TC = 400


def _tc_body(nodes_ref, a1_ref, a2_ref, cnt_ref, w0_ref, w1_ref, w2_ref,
             out_ref):
    dn = (((1,), (1,)), ((), ()))
    a1 = a1_ref[...].reshape(ROWS_TC, D)
    a2 = a2_ref[...].reshape(ROWS_TC, D)
    cnt = jnp.sum(cnt_ref[...], axis=1, keepdims=True)
    denom = jnp.maximum(cnt, 1.0)
    msg = (lax.dot_general(a1, w1_ref[...], dn,
                           preferred_element_type=jnp.float32)
           + lax.dot_general(a2, w2_ref[...], dn,
                             preferred_element_type=jnp.float32)) / denom
    self_t = lax.dot_general(nodes_ref[...], w0_ref[...], dn,
                             preferred_element_type=jnp.float32)
    out_ref[...] = jnp.maximum(self_t + msg, 0.0)


def _tc_combine(nodes, a1, a2, cnt2, w0, w1, w2):
    row_spec = pl.BlockSpec((ROWS_TC, D), lambda i: (i, 0))
    w_spec = pl.BlockSpec((D, D), lambda i: (0, 0))
    return pl.pallas_call(
        _tc_body,
        grid=(N // ROWS_TC,),
        in_specs=[row_spec,
                  pl.BlockSpec((ROWS_TC * D,), lambda i: (i,)),
                  pl.BlockSpec((ROWS_TC * D,), lambda i: (i,)),
                  pl.BlockSpec((ROWS_TC, NC), lambda i: (i, 0)),
                  w_spec, w_spec, w_spec],
        out_specs=row_spec,
        out_shape=jax.ShapeDtypeStruct((N, D), jnp.float32),
    )(nodes, a1, a2, cnt2, w0, w1, w2)


def kernel(nodes, edges, W0, W1, W2):
    eidx = edges.astype(jnp.int32).reshape(NC, NS, NBLK * BCH, CHUNK)
    a1, a2, cnt = _sc_aggregate(nodes, eidx)
    a1 = a1.reshape(NPAD * D)
    a2 = a2.reshape(NPAD * D)
    return _tc_combine(nodes, a1, a2, cnt.T, W0, W1, W2)


# speedup vs baseline: 1.0768x; 1.0125x over previous
import jax
import jax.numpy as jnp
from jax import lax
from jax.experimental import pallas as pl
from jax.experimental.pallas import tpu as pltpu
from jax.experimental.pallas import tpu_sc as plsc

N = 10000
E = 320000
D = 128

NC = 2
NS = 16
TILE_E = E // NS
CHUNK = 80
NCH = TILE_E // CHUNK
BCH = 25
NBLK = NCH // BCH
NPAD = 10240
ROWS_PT = NPAD // NS


def _sc_body(nodes_hbm, eidx_hbm,
             acc1_out, acc2_out, cnt_out,
             acc_s, cnt_s, gla, sla, glb, slb, buf0, buf1, buf2, ones,
             zcnt, sem0, sem1, sem2, semi, semj):
    c = lax.axis_index("c")
    s = lax.axis_index("s")

    for i in range(CHUNK // 16):
        ones[pl.ds(i * 16, 16)] = jnp.ones((16,), jnp.float32)

    zero16 = jnp.zeros((16,), jnp.float32)
    for r in range(CHUNK):
        for i in range(D // 16):
            buf0[r, pl.ds(i * 16, 16)] = zero16
    for i in range(ROWS_PT // 16):
        zcnt[pl.ds(i * 16, 16)] = zero16
    for r in range(ROWS_PT // CHUNK):
        pltpu.sync_copy(
            buf0, acc_s.at[pl.ds(s * ROWS_PT + r * CHUNK, CHUNK)])
    pltpu.sync_copy(zcnt, cnt_s.at[pl.ds(s * ROWS_PT, ROWS_PT)])
    plsc.subcore_barrier()

    bufs = (buf0, buf1, buf2)
    sems = (sem0, sem1, sem2)

    def proc(gl, sl):
        def gath(j, p):
            pltpu.async_copy(nodes_hbm.at[gl.at[j]], bufs[p], sems[p])

        def wait_g(p):
            pltpu.make_async_copy(
                nodes_hbm.at[gl.at[0]], bufs[p], sems[p]).wait()

        def scat(j, p):
            pltpu.sync_copy(bufs[p], acc_s.at[sl.at[j]], add=True)
            pltpu.sync_copy(ones, cnt_s.at[sl.at[j]], add=True)

        gath(0, 0)
        gath(1, 1)

        def inner(t, carry2):
            j = 3 * t
            wait_g(0)
            gath(j + 2, 2)
            scat(j, 0)
            wait_g(1)
            gath(j + 3, 0)
            scat(j + 1, 1)
            wait_g(2)
            gath(j + 4, 1)
            scat(j + 2, 2)
            return carry2

        lax.fori_loop(0, (BCH - 4) // 3, inner, 0)
        wait_g(0)
        gath(BCH - 2, 2)
        scat(BCH - 4, 0)
        wait_g(1)
        gath(BCH - 1, 0)
        scat(BCH - 3, 1)
        wait_g(2)
        scat(BCH - 2, 2)
        wait_g(0)
        scat(BCH - 1, 0)

    def load_idx(b, gl, sl, sem):
        pltpu.async_copy(eidx_hbm.at[c, s, pl.ds(b * BCH, BCH)], gl, sem)
        pltpu.async_copy(eidx_hbm.at[1 - c, s, pl.ds(b * BCH, BCH)], sl,
                         sem)

    def wait_idx(b, gl, sl, sem):
        pltpu.make_async_copy(
            eidx_hbm.at[c, s, pl.ds(0, BCH)], gl, sem).wait()
        pltpu.make_async_copy(
            eidx_hbm.at[c, s, pl.ds(0, BCH)], sl, sem).wait()

    load_idx(0, gla, sla, semi)

    def super_block(t, carry):
        ba = 2 * t
        wait_idx(ba, gla, sla, semi)
        load_idx(ba + 1, glb, slb, semj)
        proc(gla, sla)
        wait_idx(ba + 1, glb, slb, semj)
        load_idx(jnp.minimum(ba + 2, NBLK - 1), gla, sla, semi)
        proc(glb, slb)
        return carry

    lax.fori_loop(0, NBLK // 2, super_block, 0)
    pltpu.make_async_copy(eidx_hbm.at[c, s, pl.ds(0, BCH)], gla, semi).wait()
    pltpu.make_async_copy(eidx_hbm.at[c, s, pl.ds(0, BCH)], sla, semi).wait()
    plsc.subcore_barrier()

    sl_pt = pl.ds(s * ROWS_PT, ROWS_PT)

    @pl.when(c == 0)
    def _():
        pltpu.sync_copy(acc_s.at[sl_pt], acc1_out.at[sl_pt])

    @pl.when(c == 1)
    def _():
        pltpu.sync_copy(acc_s.at[sl_pt], acc2_out.at[sl_pt])

    pltpu.sync_copy(cnt_s.at[sl_pt], cnt_out.at[c, sl_pt])


_sc_aggregate = pl.kernel(
    _sc_body,
    out_type=(
        jax.ShapeDtypeStruct((NPAD, D), jnp.float32),
        jax.ShapeDtypeStruct((NPAD, D), jnp.float32),
        jax.ShapeDtypeStruct((NC, NPAD), jnp.float32),
    ),
    mesh=plsc.VectorSubcoreMesh(core_axis_name="c", subcore_axis_name="s"),
    scratch_types=[
        pltpu.VMEM_SHARED((NPAD, D), jnp.float32),
        pltpu.VMEM_SHARED((NPAD,), jnp.float32),
        pltpu.VMEM((BCH, CHUNK), jnp.int32),
        pltpu.VMEM((BCH, CHUNK), jnp.int32),
        pltpu.VMEM((BCH, CHUNK), jnp.int32),
        pltpu.VMEM((BCH, CHUNK), jnp.int32),
        pltpu.VMEM((CHUNK, D), jnp.float32),
        pltpu.VMEM((CHUNK, D), jnp.float32),
        pltpu.VMEM((CHUNK, D), jnp.float32),
        pltpu.VMEM((CHUNK,), jnp.float32),
        pltpu.VMEM((ROWS_PT,), jnp.float32),
        pltpu.SemaphoreType.DMA,
        pltpu.SemaphoreType.DMA,
        pltpu.SemaphoreType.DMA,
        pltpu.SemaphoreType.DMA,
        pltpu.SemaphoreType.DMA,
    ],
    compiler_params=pltpu.CompilerParams(use_tc_tiling_on_sc=False),
)


ROWS_TC = 400


def _tc_body(nodes_ref, a1_ref, a2_ref, cnt_ref, w0_ref, w1_ref, w2_ref,
             out_ref):
    dn = (((1,), (1,)), ((), ()))
    a1 = a1_ref[...].reshape(ROWS_TC, D)
    a2 = a2_ref[...].reshape(ROWS_TC, D)
    cnt = jnp.sum(cnt_ref[...], axis=1, keepdims=True)
    denom = jnp.maximum(cnt, 1.0)
    msg = (lax.dot_general(a1, w1_ref[...], dn,
                           preferred_element_type=jnp.float32)
           + lax.dot_general(a2, w2_ref[...], dn,
                             preferred_element_type=jnp.float32)) / denom
    self_t = lax.dot_general(nodes_ref[...], w0_ref[...], dn,
                             preferred_element_type=jnp.float32)
    out_ref[...] = jnp.maximum(self_t + msg, 0.0)


def _tc_combine(nodes, a1, a2, cnt2, w0, w1, w2):
    row_spec = pl.BlockSpec((ROWS_TC, D), lambda i: (i, 0))
    w_spec = pl.BlockSpec((D, D), lambda i: (0, 0))
    return pl.pallas_call(
        _tc_body,
        grid=(N // ROWS_TC,),
        in_specs=[row_spec,
                  pl.BlockSpec((ROWS_TC * D,), lambda i: (i,)),
                  pl.BlockSpec((ROWS_TC * D,), lambda i: (i,)),
                  pl.BlockSpec((ROWS_TC, NC), lambda i: (i, 0)),
                  w_spec, w_spec, w_spec],
        out_specs=row_spec,
        out_shape=jax.ShapeDtypeStruct((N, D), jnp.float32),
    )(nodes, a1, a2, cnt2, w0, w1, w2)


def kernel(nodes, edges, W0, W1, W2):
    eidx = edges.astype(jnp.int32).reshape(NC, NS, NBLK * BCH, CHUNK)
    a1, a2, cnt = _sc_aggregate(nodes, eidx)
    a1 = a1.reshape(NPAD * D)
    a2 = a2.reshape(NPAD * D)
    return _tc_combine(nodes, a1, a2, cnt.T, W0, W1, W2)
